# Initial kernel scaffold; baseline (speedup 1.0000x reference)
import jax
import jax.numpy as jnp
from jax.experimental import pallas as pl


def kernel(x_tag, x_text, x_class, x_num, edge_index, tag_embed, Wc, bc, Wtc, btc, Wt, bt, Wm, bm, Wn, bnm, Wl0, bl0, Wr0, g0, be0, Wl1, bl1, Wr1, g1, be1, Wl2, bl2, Wr2, g2, be2, Wh, bh):
    raise NotImplementedError("write your pallas kernel here")


# speedup vs baseline: 3.4850x; 3.4850x over previous
import functools

import jax
import jax.numpy as jnp
from jax import lax
from jax.experimental import pallas as pl
from jax.experimental.pallas import tpu as pltpu
from jax.experimental.pallas import tpu_sc as plsc

_N = 50000
_E = 800000
_HD = 64
_HH = 32
_QW = 16
_FT = 300
_NT = 150
_NN = 16
_EPS = 1e-5

_LANE = 128
_NSUB = 16
_EROWS = 392
_E_PAD = _NSUB * _EROWS * _LANE
_ROWS_TOT = _E_PAD // _LANE
_AGG_ROWS = 50048
_ZROWS = _AGG_ROWS // _NSUB
_TAIL = _N - 15 * _ZROWS

_BN = 2000
_GRID = _N // _BN

_MESH = plsc.VectorSubcoreMesh(core_axis_name="c", subcore_axis_name="s")
_f32 = jnp.float32
_bf16 = jnp.bfloat16


def _run_edges(table, src_v, dst_v, rows0, rows1, agg_sh, sem0, sem1):
    def g_start(j, buf, sem):
        pltpu.async_copy(table.at[src_v.at[j]], buf, sem)

    def g_wait(j, buf, sem):
        pltpu.make_async_copy(table.at[src_v.at[j]], buf, sem).wait()

    g_start(0, rows0, sem0)

    def body(i, carry):
        j0 = 2 * i
        g_start(j0 + 1, rows1, sem1)
        g_wait(j0, rows0, sem0)
        pltpu.sync_copy(rows0, agg_sh.at[dst_v.at[j0]], add=True)

        @pl.when(j0 + 2 < _EROWS)
        def _():
            g_start(j0 + 2, rows0, sem0)

        g_wait(j0 + 1, rows1, sem1)
        pltpu.sync_copy(rows1, agg_sh.at[dst_v.at[j0 + 1]], add=True)
        return carry

    lax.fori_loop(0, _EROWS // 2, body, 0)


def _copy_out(sh, out, s):
    @pl.when(s < _NSUB - 1)
    def _():
        pltpu.sync_copy(sh.at[pl.ds(s * _ZROWS, _ZROWS)],
                        out.at[pl.ds(s * _ZROWS, _ZROWS)])

    @pl.when(s == _NSUB - 1)
    def _():
        pltpu.sync_copy(sh.at[pl.ds(15 * _ZROWS, _TAIL)],
                        out.at[pl.ds(15 * _ZROWS, _TAIL)])


@functools.partial(
    pl.kernel,
    mesh=_MESH,
    out_type=[jax.ShapeDtypeStruct((_N, _QW), _bf16)] * 4,
    scratch_types=[
        pltpu.VMEM((_EROWS, _LANE), jnp.int32),
        pltpu.VMEM((_EROWS, _LANE), jnp.int32),
        pltpu.VMEM((_LANE, _QW), _bf16),
        pltpu.VMEM((_LANE, _QW), _bf16),
        pltpu.VMEM_SHARED((_AGG_ROWS, _QW), _bf16),
        pltpu.SemaphoreType.DMA,
        pltpu.SemaphoreType.DMA,
    ],
    compiler_params=pltpu.CompilerParams(use_tc_tiling_on_sc=False),
)
def _sc_agg(t0, t1, t2, t3, src2d, dst2d, zrows, o0, o1, o2, o3,
            src_v, dst_v, rows0, rows1, agg_sh, sem0, sem1):
    c = lax.axis_index("c")
    s = lax.axis_index("s")
    pltpu.sync_copy(src2d.at[pl.ds(s * _EROWS, _EROWS)], src_v)
    pltpu.sync_copy(dst2d.at[pl.ds(s * _EROWS, _EROWS)], dst_v)
    for tabA, tabB, outA, outB in ((t0, t1, o0, o1), (t2, t3, o2, o3)):
        pltpu.sync_copy(zrows, agg_sh.at[pl.ds(s * _ZROWS, _ZROWS)])
        plsc.subcore_barrier()

        @pl.when(c == 0)
        def _(tabA=tabA):
            _run_edges(tabA, src_v, dst_v, rows0, rows1, agg_sh, sem0, sem1)

        @pl.when(c == 1)
        def _(tabB=tabB):
            _run_edges(tabB, src_v, dst_v, rows0, rows1, agg_sh, sem0, sem1)

        plsc.subcore_barrier()

        @pl.when(c == 0)
        def _(outA=outA):
            _copy_out(agg_sh, outA, s)

        @pl.when(c == 1)
        def _(outB=outB):
            _copy_out(agg_sh, outB, s)

        plsc.subcore_barrier()


def _enc_body(tag, xt, xc, xn, temb, WcT, bc, WtcTa, WtcTb, btc, WtT, bt,
              WmAL, WmBL, bmL, WmAR, WmBR, bmR, WnL, bnL, WnR, bnR,
              h0A, h0B):
    f = _f32
    iot = lax.broadcasted_iota(jnp.int32, (_BN, _NT), 1)
    oh = (iot == tag[...]).astype(f)
    et = jnp.dot(oh, temb[...], preferred_element_type=f)
    ec = jnp.dot(xc[...], WcT[...], preferred_element_type=f) + bc[...]
    htc = jnp.maximum(jnp.dot(et, WtcTa[...], preferred_element_type=f)
                      + jnp.dot(ec, WtcTb[...], preferred_element_type=f)
                      + btc[...], 0.0)
    ht = jnp.maximum(jnp.dot(xt[...], WtT[...], preferred_element_type=f)
                     + bt[...], 0.0)
    hxL = jnp.maximum(jnp.dot(htc, WmAL[...], preferred_element_type=f)
                      + jnp.dot(ht, WmBL[...], preferred_element_type=f)
                      + bmL[...], 0.0)
    hxR = jnp.maximum(jnp.dot(htc, WmAR[...], preferred_element_type=f)
                      + jnp.dot(ht, WmBR[...], preferred_element_type=f)
                      + bmR[...], 0.0)
    hnL = jnp.maximum(jnp.dot(xn[...], WnL[...], preferred_element_type=f)
                      + bnL[...], 0.0)
    hnR = jnp.maximum(jnp.dot(xn[...], WnR[...], preferred_element_type=f)
                      + bnR[...], 0.0)
    h0A[...] = jnp.maximum(hxL + hnL, 0.0)
    h0B[...] = jnp.maximum(hxR + hnR, 0.0)


def _k2_body(a0, a1, a2, a3, deg, hA, hB,
             Wl0L, Wl1L, Wl2L, Wl3L, Wl0R, Wl1R, Wl2R, Wl3R,
             Wr00, Wr10, Wr01, Wr11, blL, blR,
             outA, outB, stA, stB):
    f = _f32
    i = pl.program_id(0)

    @pl.when(i == 0)
    def _():
        stA[...] = jnp.zeros((2, _HH), f)
        stB[...] = jnp.zeros((2, _HH), f)

    d = deg[...][:, 0:1].astype(f)
    invd = 1.0 / jnp.maximum(d, 1.0)
    m0 = a0[...].astype(f) * invd
    m1 = a1[...].astype(f) * invd
    m2 = a2[...].astype(f) * invd
    m3 = a3[...].astype(f) * invd
    a = hA[...]
    b = hB[...]
    oA = (jnp.dot(m0, Wl0L[...], preferred_element_type=f)
          + jnp.dot(m1, Wl1L[...], preferred_element_type=f)
          + jnp.dot(m2, Wl2L[...], preferred_element_type=f)
          + jnp.dot(m3, Wl3L[...], preferred_element_type=f)
          + jnp.dot(a, Wr00[...], preferred_element_type=f)
          + jnp.dot(b, Wr10[...], preferred_element_type=f) + blL[...])
    oB = (jnp.dot(m0, Wl0R[...], preferred_element_type=f)
          + jnp.dot(m1, Wl1R[...], preferred_element_type=f)
          + jnp.dot(m2, Wl2R[...], preferred_element_type=f)
          + jnp.dot(m3, Wl3R[...], preferred_element_type=f)
          + jnp.dot(a, Wr01[...], preferred_element_type=f)
          + jnp.dot(b, Wr11[...], preferred_element_type=f) + blR[...])
    outA[...] = oA
    outB[...] = oB
    stA[...] = stA[...] + jnp.concatenate(
        [jnp.sum(oA, 0, keepdims=True), jnp.sum(oA * oA, 0, keepdims=True)], 0)
    stB[...] = stB[...] + jnp.concatenate(
        [jnp.sum(oB, 0, keepdims=True), jnp.sum(oB * oB, 0, keepdims=True)], 0)


def _k4_body(outA, outB, hA, hB, scA, shA, scB, shB,
             nA, nB, q0, q1, q2, q3):
    tA = jnp.maximum(outA[...] * scA[...] + shA[...], 0.0) + hA[...]
    tB = jnp.maximum(outB[...] * scB[...] + shB[...], 0.0) + hB[...]
    nA[...] = tA
    nB[...] = tB
    q0[...] = tA[:, 0:_QW].astype(_bf16)
    q1[...] = tA[:, _QW:_HH].astype(_bf16)
    q2[...] = tB[:, 0:_QW].astype(_bf16)
    q3[...] = tB[:, _QW:_HH].astype(_bf16)


def _head_body(hA, hB, WhA, WhB, bh, lg):
    f = _f32
    lg[...] = (jnp.dot(hA[...], WhA[...], preferred_element_type=f)
               + jnp.dot(hB[...], WhB[...], preferred_element_type=f)
               + bh[...])


def _row_spec(w):
    return pl.BlockSpec((_BN, w), lambda i: (i, 0))


def _full_spec(a, b):
    return pl.BlockSpec((a, b), lambda i: (0, 0))


def kernel(x_tag, x_text, x_class, x_num, edge_index, tag_embed, Wc, bc, Wtc,
           btc, Wt, bt, Wm, bm, Wn, bnm, Wl0, bl0, Wr0, g0, be0, Wl1, bl1,
           Wr1, g1, be1, Wl2, bl2, Wr2, g2, be2, Wh, bh):
    f = _f32
    src = edge_index[0].astype(jnp.int32)
    dst = edge_index[1].astype(jnp.int32)
    pad = _E_PAD - _E
    srcp = jnp.concatenate([src, jnp.zeros((pad,), jnp.int32)])
    dstp = jnp.concatenate([dst, jnp.full((pad,), _N, jnp.int32)])
    src2d = srcp.reshape(_ROWS_TOT, _LANE)
    dst2d = dstp.reshape(_ROWS_TOT, _LANE)
    zrows = jnp.zeros((_ZROWS, _QW), _bf16)

    tag2 = x_tag.astype(jnp.int32).reshape(_N, 1)
    WtcT = Wtc.T
    WmT = Wm.T
    WnT = Wn.T
    enc_in = (
        tag2, x_text, x_class, x_num,
        tag_embed.astype(f), Wc.T, bc.reshape(1, _HD),
        WtcT[:_HD], WtcT[_HD:], btc.reshape(1, _HD),
        Wt.T, bt.reshape(1, _HD),
        WmT[:_HD, :_HH], WmT[_HD:, :_HH], bm[:_HH].reshape(1, _HH),
        WmT[:_HD, _HH:], WmT[_HD:, _HH:], bm[_HH:].reshape(1, _HH),
        WnT[:, :_HH], bnm[:_HH].reshape(1, _HH),
        WnT[:, _HH:], bnm[_HH:].reshape(1, _HH),
    )
    enc_specs = [
        _row_spec(1), _row_spec(_FT), _row_spec(_FT), _row_spec(_NN),
        _full_spec(_NT, _HD), _full_spec(_FT, _HD), _full_spec(1, _HD),
        _full_spec(_HD, _HD), _full_spec(_HD, _HD), _full_spec(1, _HD),
        _full_spec(_FT, _HD), _full_spec(1, _HD),
        _full_spec(_HD, _HH), _full_spec(_HD, _HH), _full_spec(1, _HH),
        _full_spec(_HD, _HH), _full_spec(_HD, _HH), _full_spec(1, _HH),
        _full_spec(_NN, _HH), _full_spec(1, _HH),
        _full_spec(_NN, _HH), _full_spec(1, _HH),
    ]
    hA, hB = pl.pallas_call(
        _enc_body,
        grid=(_GRID,),
        in_specs=enc_specs,
        out_specs=[_row_spec(_HH)] * 2,
        out_shape=[jax.ShapeDtypeStruct((_N, _HH), f)] * 2,
        compiler_params=pltpu.CompilerParams(
            dimension_semantics=("parallel",)),
    )(*enc_in)

    z64 = jnp.zeros((_HD, _HD), f)
    z1 = jnp.zeros((_HD,), f)
    WlTs = jnp.stack([z64, Wl0.T, Wl1.T, Wl2.T])
    WrTs = jnp.stack([z64, Wr0.T, Wr1.T, Wr2.T])
    bls = jnp.stack([z1, bl0, bl1, bl2])
    gs = jnp.stack([z1, g0, g1, g2])
    bes = jnp.stack([z1, be0, be1, be2])
    ts = jnp.arange(4)

    def step(carry, w):
        hA, hB, q0, q1, q2, q3, degb = carry
        t, WlT, WrT, bl, g, be = w
        a0, a1, a2, a3 = _sc_agg(q0, q1, q2, q3, src2d, dst2d, zrows)
        degb = jnp.where(t == 0, a0, degb)
        k2_in = (
            a0, a1, a2, a3, degb, hA, hB,
            WlT[0:16, :_HH], WlT[16:32, :_HH],
            WlT[32:48, :_HH], WlT[48:64, :_HH],
            WlT[0:16, _HH:], WlT[16:32, _HH:],
            WlT[32:48, _HH:], WlT[48:64, _HH:],
            WrT[:_HH, :_HH], WrT[_HH:, :_HH], WrT[:_HH, _HH:], WrT[_HH:, _HH:],
            bl[:_HH].reshape(1, _HH), bl[_HH:].reshape(1, _HH),
        )
        k2_specs = (
            [_row_spec(_QW)] * 5 + [_row_spec(_HH)] * 2
            + [_full_spec(_QW, _HH)] * 8
            + [_full_spec(_HH, _HH)] * 4
            + [_full_spec(1, _HH)] * 2
        )
        outA, outB, stA, stB = pl.pallas_call(
            _k2_body,
            grid=(_GRID,),
            in_specs=k2_specs,
            out_specs=[_row_spec(_HH), _row_spec(_HH),
                       pl.BlockSpec((2, _HH), lambda i: (0, 0)),
                       pl.BlockSpec((2, _HH), lambda i: (0, 0))],
            out_shape=[jax.ShapeDtypeStruct((_N, _HH), f)] * 2
            + [jax.ShapeDtypeStruct((2, _HH), f)] * 2,
            compiler_params=pltpu.CompilerParams(
                dimension_semantics=("arbitrary",)),
        )(*k2_in)

        muA = stA[0] / _N
        varA = stA[1] / _N - muA * muA
        scA = (g[:_HH] / jnp.sqrt(varA + _EPS)).reshape(1, _HH)
        shA = (be[:_HH] - muA * scA[0]).reshape(1, _HH)
        muB = stB[0] / _N
        varB = stB[1] / _N - muB * muB
        scB = (g[_HH:] / jnp.sqrt(varB + _EPS)).reshape(1, _HH)
        shB = (be[_HH:] - muB * scB[0]).reshape(1, _HH)

        hA, hB, q0, q1, q2, q3 = pl.pallas_call(
            _k4_body,
            grid=(_GRID,),
            in_specs=[_row_spec(_HH)] * 4 + [_full_spec(1, _HH)] * 4,
            out_specs=[_row_spec(_HH)] * 2 + [_row_spec(_QW)] * 4,
            out_shape=[jax.ShapeDtypeStruct((_N, _HH), f)] * 2
            + [jax.ShapeDtypeStruct((_N, _QW), _bf16)] * 4,
            compiler_params=pltpu.CompilerParams(
                dimension_semantics=("parallel",)),
        )(outA, outB, hA, hB, scA, shA, scB, shB)
        return (hA, hB, q0, q1, q2, q3, degb), None

    ones_q = jnp.ones((_N, _QW), _bf16)
    (hA, hB, q0, q1, q2, q3, _), _ = lax.scan(
        step, (hA, hB, ones_q, ones_q, ones_q, ones_q, ones_q),
        (ts, WlTs, WrTs, bls, gs, bes))

    WhT = Wh.T.astype(f)
    logits = pl.pallas_call(
        _head_body,
        grid=(_GRID,),
        in_specs=[_row_spec(_HH)] * 2
        + [_full_spec(_HH, 2), _full_spec(_HH, 2), _full_spec(1, 2)],
        out_specs=[_row_spec(2)],
        out_shape=[jax.ShapeDtypeStruct((_N, 2), f)],
        compiler_params=pltpu.CompilerParams(
            dimension_semantics=("parallel",)),
    )(hA, hB, WhT[:_HH], WhT[_HH:], bh.reshape(1, 2))[0]

    return logits
